# baseline (speedup 1.0000x reference)
import functools

import jax
import jax.numpy as jnp
from jax import lax
from jax.experimental import pallas as pl
from jax.experimental.pallas import tpu as pltpu
from jax.experimental.pallas import tpu_sc as plsc

N_NODES = 50000
N_EDGES = 800000
U = 16
L = 16
NC, NS = 2, 16
NW = NC * NS
S_NODE = 1568
N_PAD = NW * S_NODE
E_W = N_EDGES // NW
EK = 128
N_FULL = E_W // EK
E_TAIL = E_W - N_FULL * EK

_mesh = plsc.VectorSubcoreMesh(core_axis_name="c", subcore_axis_name="s")


def _wid():
    return lax.axis_index("s") * NC + lax.axis_index("c")


def _iota():
    return lax.iota(jnp.int32, L)


def _vperm(x, idx):
    return x.at[idx].get(mode="promise_in_bounds")


@functools.partial(
    pl.kernel,
    mesh=_mesh,
    compiler_params=pltpu.CompilerParams(needs_layout_passes=False, use_tc_tiling_on_sc=False),
    out_type=jax.ShapeDtypeStruct((N_PAD, U), jnp.float32),
    scratch_types=[
        pltpu.VMEM((112, U), jnp.float32),
        pltpu.VMEM((L,), jnp.float32),
    ],
)
def _k1(unary_hbm, wu_hbm, u_hbm, rowbuf, wubuf):
    wid = _wid()
    io = _iota()
    wubuf[...] = jnp.zeros((L,), jnp.float32)
    pltpu.sync_copy(wu_hbm, wubuf.at[pl.ds(0, 4)])
    wuv = wubuf[...]
    wulane = _vperm(wuv, io >> 1)
    evenlane = (io & 1) == 0
    active = io < 8
    sgn_a = jnp.where(active & evenlane, -1.0, jnp.where(active, 1.0, 0.0))
    sgn_d = sgn_a
    perm = jnp.where(active, io ^ 1, io)
    base = wid * S_NODE

    def row(r, _):
        x = rowbuf[r]
        e = jnp.exp(sgn_a * x)
        esw = _vperm(e, perm)
        delta = sgn_d * wulane * e / (e + esw)
        rowbuf[r] = x + delta
        return 0

    CH = 112
    TAILBASE = (N_NODES // CH) * CH

    def chunk(c, _):
        rbase = base + c * CH

        @pl.when(rbase + CH <= N_NODES)
        def _():
            pltpu.sync_copy(unary_hbm.at[pl.ds(rbase, CH)], rowbuf)
            lax.fori_loop(0, CH, row, 0)
            pltpu.sync_copy(rowbuf, u_hbm.at[pl.ds(rbase, CH)])

        @pl.when(rbase == TAILBASE)
        def _():
            nt = N_NODES - TAILBASE
            pltpu.sync_copy(unary_hbm.at[pl.ds(rbase, nt)],
                            rowbuf.at[pl.ds(0, nt)])
            lax.fori_loop(0, nt, row, 0)
            pltpu.sync_copy(rowbuf.at[pl.ds(0, nt)],
                            u_hbm.at[pl.ds(rbase, nt)])

        return 0

    lax.fori_loop(0, S_NODE // CH, chunk, 0)


@functools.partial(
    pl.kernel,
    mesh=_mesh,
    compiler_params=pltpu.CompilerParams(needs_layout_passes=False, use_tc_tiling_on_sc=False),
    out_type=[
        jax.ShapeDtypeStruct((N_EDGES,), jnp.float32),
        jax.ShapeDtypeStruct((NW, N_PAD), jnp.int32),
        jax.ShapeDtypeStruct((NW, N_PAD), jnp.int32),
    ],
    scratch_types=[
        pltpu.VMEM((N_PAD,), jnp.int32),
        pltpu.VMEM((N_PAD,), jnp.int32),
        pltpu.VMEM((2, EK), jnp.int32),
        pltpu.VMEM((2, EK), jnp.int32),
        pltpu.VMEM((2, EK), jnp.float32),
        pltpu.VMEM((2, EK), jnp.float32),
        pltpu.VMEM((2, EK, U), jnp.float32),
        pltpu.VMEM((2, EK, U), jnp.float32),
        pltpu.VMEM((L,), jnp.float32),
        pltpu.SemaphoreType.DMA((2,)),
        pltpu.SemaphoreType.DMA((2,)),
        pltpu.SemaphoreType.DMA((2,)),
    ],
)
def _k2(u_hbm, b_hbm, i1_hbm, i2_hbm, wb_hbm, bout_hbm, t1_hbm, t2_hbm,
        t1, t2, i1c, i2c, bc, boutc, rows1, rows2, wbbuf,
        semi, semg, semo):
    wid = _wid()
    io = _iota()
    pltpu.sync_copy(wb_hbm, wbbuf)
    wblv = wbbuf[...]
    ebase = wid * E_W

    def initt(i, _):
        neg1 = jnp.full((L,), -1, jnp.int32)
        t1[pl.ds(i * L, L)] = neg1
        t2[pl.ds(i * L, L)] = neg1
        return 0

    lax.fori_loop(0, N_PAD // L, initt, 0)

    def in_copies(c, s):
        off = ebase + c * EK
        return (
            pltpu.make_async_copy(i1_hbm.at[pl.ds(off, EK)], i1c.at[s], semi.at[s]),
            pltpu.make_async_copy(i2_hbm.at[pl.ds(off, EK)], i2c.at[s], semi.at[s]),
            pltpu.make_async_copy(b_hbm.at[pl.ds(off, EK)], bc.at[s], semi.at[s]),
        )

    def gather_copies(s):
        return (
            pltpu.make_async_copy(u_hbm.at[i1c.at[s]], rows1.at[s], semg.at[s]),
            pltpu.make_async_copy(u_hbm.at[i2c.at[s]], rows2.at[s], semg.at[s]),
        )

    def out_copy(c, s):
        off = ebase + c * EK
        return pltpu.make_async_copy(boutc.at[s], bout_hbm.at[pl.ds(off, EK)],
                                     semo.at[s])

    def do_group(t, off, n_valid, s):
        b16 = bc[s, pl.ds(t * L, L)]
        eb = jnp.exp(-b16)
        rowidx = t * L + io
        acc = jnp.zeros((L,), jnp.float32)
        for c in range(U):
            colidx = jnp.full((L,), c, jnp.int32)
            v1 = plsc.load_gather(rows1.at[s], [rowidx, colidx])
            v2 = plsc.load_gather(rows2.at[s], [rowidx, colidx])
            d = jnp.exp(-v1) + eb + jnp.exp(v2)
            acc = acc + wblv[c] / d
        boutc[s, pl.ds(t * L, L)] = b16 - eb * acc
        eids = off + t * L + io
        for idxc, tab in ((i1c, t1), (i2c, t2)):
            nid = idxc[s, pl.ds(t * L, L)]
            key = nid * L + io
            ks, vs = plsc.sort_key_val(key, eids)
            nxt = _vperm(ks, jnp.minimum(io + 1, L - 1))
            last = ((ks >> 4) != (nxt >> 4)) | (io == L - 1)
            vmask = (ks & (L - 1)) < (n_valid - t * L)
            plsc.store_scatter(tab, [ks >> 4], vs, mask=last & vmask)

    for d in in_copies(0, 0) + in_copies(1, 1):
        d.start()
    for d in in_copies(0, 0):
        d.wait()
    for d in gather_copies(0):
        d.start()

    def pipe_chunk(c, _):
        s = lax.rem(c, 2)
        sn = 1 - s
        for d in gather_copies(s):
            d.wait()

        @pl.when(c + 1 < N_FULL)
        def _():
            for d in in_copies(c + 1, sn):
                d.wait()
            for d in gather_copies(sn):
                d.start()

        @pl.when(c >= 2)
        def _():
            out_copy(c - 2, s).wait()

        off = ebase + c * EK

        def grp(t, _):
            do_group(2 * t, off, EK, s)
            do_group(2 * t + 1, off, EK, s)
            return 0

        lax.fori_loop(0, EK // L // 2, grp, 0)
        out_copy(c, s).start()

        @pl.when(c + 2 < N_FULL)
        def _():
            for d in in_copies(c + 2, s):
                d.start()

        return 0

    lax.fori_loop(0, N_FULL, pipe_chunk, 0)
    out_copy(N_FULL - 2, (N_FULL - 2) % 2).wait()
    out_copy(N_FULL - 1, (N_FULL - 1) % 2).wait()

    toff = ebase + N_FULL * EK
    pltpu.sync_copy(i1_hbm.at[pl.ds(toff, E_TAIL)], i1c.at[0, pl.ds(0, E_TAIL)])
    pltpu.sync_copy(i2_hbm.at[pl.ds(toff, E_TAIL)], i2c.at[0, pl.ds(0, E_TAIL)])
    pltpu.sync_copy(b_hbm.at[pl.ds(toff, E_TAIL)], bc.at[0, pl.ds(0, E_TAIL)])
    pltpu.sync_copy(u_hbm.at[i1c.at[0]], rows1.at[0])
    pltpu.sync_copy(u_hbm.at[i2c.at[0]], rows2.at[0])

    def tgrp(t, _):
        do_group(t, toff, E_TAIL, 0)
        return 0

    lax.fori_loop(0, (E_TAIL + L - 1) // L, tgrp, 0)
    pltpu.sync_copy(boutc.at[0, pl.ds(0, E_TAIL)],
                    bout_hbm.at[pl.ds(toff, E_TAIL)])

    pltpu.sync_copy(t1, t1_hbm.at[wid])
    pltpu.sync_copy(t2, t2_hbm.at[wid])


GK = 112


@functools.partial(
    pl.kernel,
    mesh=_mesh,
    compiler_params=pltpu.CompilerParams(needs_layout_passes=False, use_tc_tiling_on_sc=False),
    out_type=jax.ShapeDtypeStruct((N_PAD, U), jnp.float32),
    scratch_types=[
        pltpu.VMEM((S_NODE,), jnp.int32),
        pltpu.VMEM((S_NODE,), jnp.int32),
        pltpu.VMEM((2, S_NODE), jnp.int32),
        pltpu.VMEM((S_NODE // GK, GK), jnp.int32),
        pltpu.VMEM((S_NODE // GK, GK), jnp.int32),
        pltpu.VMEM((S_NODE // GK, GK), jnp.int32),
        pltpu.VMEM((S_NODE // GK, GK), jnp.int32),
        pltpu.VMEM((S_NODE,), jnp.float32),
        pltpu.VMEM((S_NODE,), jnp.float32),
        pltpu.VMEM((S_NODE, U), jnp.float32),
        pltpu.VMEM((S_NODE, U), jnp.float32),
        pltpu.VMEM((S_NODE, U), jnp.float32),
        pltpu.VMEM((L,), jnp.float32),
        pltpu.SemaphoreType.DMA,
        pltpu.SemaphoreType.DMA,
        pltpu.SemaphoreType.DMA((2,)),
        pltpu.SemaphoreType.DMA((2,)),
        pltpu.SemaphoreType.DMA((2,)),
        pltpu.SemaphoreType.DMA((2,)),
        pltpu.SemaphoreType.DMA((2,)),
    ],
)
def _k3(u_hbm, b_hbm, i1_hbm, i2_hbm, wb_hbm, t1_hbm, t2_hbm, uout_hbm,
        acc1, acc2, tmp, ecl1, ecl2, jfar1, jfar2, bw1, bw2,
        rowsf1, rowsf2, ubuf, wbbuf,
        semus, sema, semm, semfb1, semfb2, semu1, semu2):
    wid = _wid()
    base = wid * S_NODE
    ubuf_cp = pltpu.make_async_copy(u_hbm.at[pl.ds(base, S_NODE)], ubuf, semus)
    ubuf_cp.start()
    pltpu.sync_copy(wb_hbm, wbbuf)
    wbl = wbbuf[...]

    def merge(acc, tab_hbm):
        acc_cp = pltpu.make_async_copy(tab_hbm.at[0, pl.ds(base, S_NODE)],
                                       acc, sema)
        acc_cp.start()

        def tcopy(t, s):
            return pltpu.make_async_copy(tab_hbm.at[t, pl.ds(base, S_NODE)],
                                         tmp.at[s], semm.at[s])

        tcopy(1, 1).start()
        acc_cp.wait()

        def step(t, _):
            s = lax.rem(t, 2)

            @pl.when(t + 1 < NW)
            def _():
                tcopy(t + 1, 1 - s).start()

            tcopy(t, s).wait()

            def mx(k, _):
                sl = pl.ds(k * L, L)
                acc[sl] = jnp.maximum(acc[sl], tmp[s, sl])
                return 0

            lax.fori_loop(0, S_NODE // L, mx, 0)
            return 0

        lax.fori_loop(1, NW, step, 0)

    def clamp_into(acc, ecl):
        def clampk(k, _):
            def clampc(cc, _):
                ecl[k, pl.ds(cc * L, L)] = jnp.clip(
                    acc[pl.ds(k * GK + cc * L, L)], 0, N_EDGES - 1)
                return 0

            lax.fori_loop(0, GK // L, clampc, 0)
            return 0

        lax.fori_loop(0, S_NODE // GK, clampk, 0)

    merge(acc1, t1_hbm)
    clamp_into(acc1, ecl1)
    merge(acc2, t2_hbm)
    clamp_into(acc2, ecl2)

    NGK = S_NODE // GK

    def farb_copies(k, s, ecl, far_hbm, jfar, bwbuf, sem):
        sl = pl.ds(k * GK, GK)
        return (
            pltpu.make_async_copy(far_hbm.at[ecl.at[k]], jfar.at[k], sem.at[s]),
            pltpu.make_async_copy(b_hbm.at[ecl.at[k]], bwbuf.at[sl], sem.at[s]),
        )

    def u_copy(k, s, jfar, rowsbuf, sem):
        sl = pl.ds(k * GK, GK)
        return pltpu.make_async_copy(u_hbm.at[jfar.at[k]], rowsbuf.at[sl],
                                     sem.at[s])

    def fb1(k, s):
        return farb_copies(k, s, ecl1, i2_hbm, jfar1, bw1, semfb1)

    def fb2(k, s):
        return farb_copies(k, s, ecl2, i1_hbm, jfar2, bw2, semfb2)

    def uc1(k, s):
        return u_copy(k, s, jfar1, rowsf1, semu1)

    def uc2(k, s):
        return u_copy(k, s, jfar2, rowsf2, semu2)

    for d in fb1(0, 0) + fb2(0, 0) + fb1(1, 1) + fb2(1, 1):
        d.start()

    def gloop(k, _):
        s = lax.rem(k, 2)
        for d in fb1(k, s) + fb2(k, s):
            d.wait()

        @pl.when(k + 2 < NGK)
        def _():
            for d in fb1(k + 2, s) + fb2(k + 2, s):
                d.start()

        @pl.when(k >= 2)
        def _():
            uc1(k - 2, s).wait()
            uc2(k - 2, s).wait()

        uc1(k, s).start()
        uc2(k, s).start()
        return 0

    lax.fori_loop(0, NGK, gloop, 0)
    for k in (NGK - 2, NGK - 1):
        uc1(k, k % 2).wait()
        uc2(k, k % 2).wait()
    ubuf_cp.wait()

    io = _iota()

    def group(g, _):
        sl = pl.ds(g * L, L)
        m1 = acc1[sl] >= 0
        m2 = acc2[sl] >= 0
        eb1 = jnp.exp(-bw1[sl])
        eb2 = jnp.exp(-bw2[sl])
        rowvec = g * L + io
        for c in range(U):
            colvec = jnp.full((L,), c, jnp.int32)
            x = plsc.load_gather(ubuf, [rowvec, colvec])
            u2 = plsc.load_gather(rowsf1, [rowvec, colvec])
            u1 = plsc.load_gather(rowsf2, [rowvec, colvec])
            exn = jnp.exp(-x)
            exp_ = jnp.exp(x)
            den1 = exn + eb1 + jnp.exp(u2)
            d1 = jnp.where(m1, -(wbl[c] * exn / den1), 0.0)
            den2 = jnp.exp(-u1) + eb2 + exp_
            d2 = jnp.where(m2, wbl[c] * exp_ / den2, 0.0)
            plsc.store_scatter(ubuf, [rowvec, colvec], x + d1 + d2)
        return 0

    lax.fori_loop(0, S_NODE // L, group, 0)
    pltpu.sync_copy(ubuf, uout_hbm.at[pl.ds(base, S_NODE)])


def kernel(unary, binary, index1, index2, w_unary, w_binary):
    u_pad = _k1(unary, w_unary)
    bout, t1, t2 = _k2(u_pad, binary, index1, index2, w_binary)
    uout_pad = _k3(u_pad, binary, index1, index2, w_binary, t1, t2)
    return uout_pad[:N_NODES], bout


# speedup vs baseline: 1.4585x; 1.4585x over previous
import functools

import jax
import jax.numpy as jnp
from jax import lax
from jax.experimental import pallas as pl
from jax.experimental.pallas import tpu as pltpu
from jax.experimental.pallas import tpu_sc as plsc

N_NODES = 50000
N_EDGES = 800000
U = 16
L = 16
NC, NS = 2, 16
NW = NC * NS
S_NODE = 1568
N_PAD = NW * S_NODE
E_W = N_EDGES // NW
EK = 128
N_FULL = E_W // EK
E_TAIL = E_W - N_FULL * EK

_mesh = plsc.VectorSubcoreMesh(core_axis_name="c", subcore_axis_name="s")


def _wid():
    return lax.axis_index("s") * NC + lax.axis_index("c")


def _iota():
    return lax.iota(jnp.int32, L)


def _vperm(x, idx):
    return x.at[idx].get(mode="promise_in_bounds")


@functools.partial(
    pl.kernel,
    mesh=_mesh,
    compiler_params=pltpu.CompilerParams(needs_layout_passes=False, use_tc_tiling_on_sc=False),
    out_type=jax.ShapeDtypeStruct((N_PAD, U), jnp.float32),
    scratch_types=[
        pltpu.VMEM((2, 112, U), jnp.float32),
        pltpu.VMEM((2, 112, U), jnp.float32),
        pltpu.VMEM((L,), jnp.float32),
        pltpu.SemaphoreType.DMA((2,)),
        pltpu.SemaphoreType.DMA((2,)),
    ],
)
def _k1(unary_hbm, wu_hbm, u_hbm, inbuf, outbuf, wubuf, semi, semo):
    wid = _wid()
    io = _iota()
    wubuf[...] = jnp.zeros((L,), jnp.float32)
    pltpu.sync_copy(wu_hbm, wubuf.at[pl.ds(0, 4)])
    wuv = wubuf[...]
    wulane = _vperm(wuv, io >> 1)
    evenlane = (io & 1) == 0
    active = io < 8
    sgn_a = jnp.where(active & evenlane, -1.0, jnp.where(active, 1.0, 0.0))
    sgn_d = sgn_a
    perm = jnp.where(active, io ^ 1, io)
    base = wid * S_NODE

    CH = 112
    NCH = S_NODE // CH

    def start_of(c):
        return jnp.minimum(base + c * CH, N_NODES - CH)

    def in_cp(c, s):
        return pltpu.make_async_copy(unary_hbm.at[pl.ds(start_of(c), CH)],
                                     inbuf.at[s], semi.at[s])

    def out_cp(c, s):
        return pltpu.make_async_copy(outbuf.at[s],
                                     u_hbm.at[pl.ds(start_of(c), CH)],
                                     semo.at[s])

    in_cp(0, 0).start()
    in_cp(1, 1).start()

    def chunk(c, _):
        s = lax.rem(c, 2)
        in_cp(c, s).wait()

        @pl.when(c >= 2)
        def _():
            out_cp(c - 2, s).wait()

        def row(r, _):
            x = inbuf[s, r]
            e = jnp.exp(sgn_a * x)
            esw = _vperm(e, perm)
            delta = sgn_d * wulane * e / (e + esw)
            outbuf[s, r] = x + delta
            return 0

        lax.fori_loop(0, CH, row, 0)
        out_cp(c, s).start()

        @pl.when(c + 2 < NCH)
        def _():
            in_cp(c + 2, s).start()

        return 0

    lax.fori_loop(0, NCH, chunk, 0)
    out_cp(NCH - 2, (NCH - 2) % 2).wait()
    out_cp(NCH - 1, (NCH - 1) % 2).wait()


@functools.partial(
    pl.kernel,
    mesh=_mesh,
    compiler_params=pltpu.CompilerParams(needs_layout_passes=False, use_tc_tiling_on_sc=False),
    out_type=[
        jax.ShapeDtypeStruct((N_EDGES,), jnp.float32),
        jax.ShapeDtypeStruct((NW, N_PAD), jnp.int32),
        jax.ShapeDtypeStruct((NW, N_PAD), jnp.int32),
    ],
    scratch_types=[
        pltpu.VMEM((N_PAD,), jnp.int32),
        pltpu.VMEM((N_PAD,), jnp.int32),
        pltpu.VMEM((2, EK), jnp.int32),
        pltpu.VMEM((2, EK), jnp.int32),
        pltpu.VMEM((2, EK), jnp.float32),
        pltpu.VMEM((2, EK), jnp.float32),
        pltpu.VMEM((2, EK, U), jnp.float32),
        pltpu.VMEM((2, EK, U), jnp.float32),
        pltpu.VMEM((L,), jnp.float32),
        pltpu.SemaphoreType.DMA((2,)),
        pltpu.SemaphoreType.DMA((2,)),
        pltpu.SemaphoreType.DMA((2,)),
    ],
)
def _k2(u_hbm, b_hbm, i1_hbm, i2_hbm, wb_hbm, bout_hbm, t1_hbm, t2_hbm,
        t1, t2, i1c, i2c, bc, boutc, rows1, rows2, wbbuf,
        semi, semg, semo):
    wid = _wid()
    io = _iota()
    pltpu.sync_copy(wb_hbm, wbbuf)
    wblv = wbbuf[...]
    ebase = wid * E_W

    def initt(i, _):
        neg1 = jnp.full((L,), -1, jnp.int32)
        t1[pl.ds(i * L, L)] = neg1
        t2[pl.ds(i * L, L)] = neg1
        return 0

    lax.fori_loop(0, N_PAD // L, initt, 0)

    def in_copies(c, s):
        off = ebase + c * EK
        return (
            pltpu.make_async_copy(i1_hbm.at[pl.ds(off, EK)], i1c.at[s], semi.at[s]),
            pltpu.make_async_copy(i2_hbm.at[pl.ds(off, EK)], i2c.at[s], semi.at[s]),
            pltpu.make_async_copy(b_hbm.at[pl.ds(off, EK)], bc.at[s], semi.at[s]),
        )

    def gather_copies(s):
        return (
            pltpu.make_async_copy(u_hbm.at[i1c.at[s]], rows1.at[s], semg.at[s]),
            pltpu.make_async_copy(u_hbm.at[i2c.at[s]], rows2.at[s], semg.at[s]),
        )

    def out_copy(c, s):
        off = ebase + c * EK
        return pltpu.make_async_copy(boutc.at[s], bout_hbm.at[pl.ds(off, EK)],
                                     semo.at[s])

    def do_group(t, off, n_valid, s):
        b16 = bc[s, pl.ds(t * L, L)]
        eb = jnp.exp(-b16)
        rowidx = t * L + io
        acc = jnp.zeros((L,), jnp.float32)
        for c in range(U):
            colidx = jnp.full((L,), c, jnp.int32)
            v1 = plsc.load_gather(rows1.at[s], [rowidx, colidx])
            v2 = plsc.load_gather(rows2.at[s], [rowidx, colidx])
            d = jnp.exp(-v1) + eb + jnp.exp(v2)
            acc = acc + wblv[c] / d
        boutc[s, pl.ds(t * L, L)] = b16 - eb * acc
        eids = off + t * L + io
        for idxc, tab in ((i1c, t1), (i2c, t2)):
            nid = idxc[s, pl.ds(t * L, L)]
            key = nid * L + io
            ks, vs = plsc.sort_key_val(key, eids)
            nxt = _vperm(ks, jnp.minimum(io + 1, L - 1))
            last = ((ks >> 4) != (nxt >> 4)) | (io == L - 1)
            vmask = (ks & (L - 1)) < (n_valid - t * L)
            plsc.store_scatter(tab, [ks >> 4], vs, mask=last & vmask)

    for d in in_copies(0, 0) + in_copies(1, 1):
        d.start()
    for d in in_copies(0, 0):
        d.wait()
    for d in gather_copies(0):
        d.start()

    def pipe_chunk(c, _):
        s = lax.rem(c, 2)
        sn = 1 - s
        for d in gather_copies(s):
            d.wait()

        @pl.when(c + 1 < N_FULL)
        def _():
            for d in in_copies(c + 1, sn):
                d.wait()
            for d in gather_copies(sn):
                d.start()

        @pl.when(c >= 2)
        def _():
            out_copy(c - 2, s).wait()

        off = ebase + c * EK

        def grp(t, _):
            do_group(t, off, EK, s)
            return 0

        lax.fori_loop(0, EK // L, grp, 0)
        out_copy(c, s).start()

        @pl.when(c + 2 < N_FULL)
        def _():
            for d in in_copies(c + 2, s):
                d.start()

        return 0

    lax.fori_loop(0, N_FULL, pipe_chunk, 0)
    out_copy(N_FULL - 2, (N_FULL - 2) % 2).wait()
    out_copy(N_FULL - 1, (N_FULL - 1) % 2).wait()

    toff = ebase + N_FULL * EK
    pltpu.sync_copy(i1_hbm.at[pl.ds(toff, E_TAIL)], i1c.at[0, pl.ds(0, E_TAIL)])
    pltpu.sync_copy(i2_hbm.at[pl.ds(toff, E_TAIL)], i2c.at[0, pl.ds(0, E_TAIL)])
    pltpu.sync_copy(b_hbm.at[pl.ds(toff, E_TAIL)], bc.at[0, pl.ds(0, E_TAIL)])
    pltpu.sync_copy(u_hbm.at[i1c.at[0]], rows1.at[0])
    pltpu.sync_copy(u_hbm.at[i2c.at[0]], rows2.at[0])

    def tgrp(t, _):
        do_group(t, toff, E_TAIL, 0)
        return 0

    lax.fori_loop(0, (E_TAIL + L - 1) // L, tgrp, 0)
    pltpu.sync_copy(boutc.at[0, pl.ds(0, E_TAIL)],
                    bout_hbm.at[pl.ds(toff, E_TAIL)])

    pltpu.sync_copy(t1, t1_hbm.at[wid])
    pltpu.sync_copy(t2, t2_hbm.at[wid])


GK = 112


@functools.partial(
    pl.kernel,
    mesh=_mesh,
    compiler_params=pltpu.CompilerParams(needs_layout_passes=False, use_tc_tiling_on_sc=False),
    out_type=jax.ShapeDtypeStruct((N_PAD, U), jnp.float32),
    scratch_types=[
        pltpu.VMEM((S_NODE,), jnp.int32),
        pltpu.VMEM((S_NODE,), jnp.int32),
        pltpu.VMEM((4, S_NODE), jnp.int32),
        pltpu.VMEM((S_NODE // GK, GK), jnp.int32),
        pltpu.VMEM((S_NODE // GK, GK), jnp.int32),
        pltpu.VMEM((S_NODE // GK, GK), jnp.int32),
        pltpu.VMEM((S_NODE // GK, GK), jnp.int32),
        pltpu.VMEM((S_NODE,), jnp.float32),
        pltpu.VMEM((S_NODE,), jnp.float32),
        pltpu.VMEM((S_NODE, U), jnp.float32),
        pltpu.VMEM((S_NODE, U), jnp.float32),
        pltpu.VMEM((S_NODE, U), jnp.float32),
        pltpu.VMEM((L,), jnp.float32),
        pltpu.SemaphoreType.DMA,
        pltpu.SemaphoreType.DMA,
        pltpu.SemaphoreType.DMA((4,)),
        pltpu.SemaphoreType.DMA((2,)),
        pltpu.SemaphoreType.DMA((2,)),
        pltpu.SemaphoreType.DMA((2,)),
        pltpu.SemaphoreType.DMA((2,)),
    ],
)
def _k3(u_hbm, b_hbm, i1_hbm, i2_hbm, wb_hbm, t1_hbm, t2_hbm, uout_hbm,
        acc1, acc2, tmp, ecl1, ecl2, jfar1, jfar2, bw1, bw2,
        rowsf1, rowsf2, ubuf, wbbuf,
        semus, sema, semm, semfb1, semfb2, semu1, semu2):
    wid = _wid()
    base = wid * S_NODE
    ubuf_cp = pltpu.make_async_copy(u_hbm.at[pl.ds(base, S_NODE)], ubuf, semus)
    ubuf_cp.start()
    pltpu.sync_copy(wb_hbm, wbbuf)
    wbl = wbbuf[...]

    def merge(acc, tab_hbm):
        acc_cp = pltpu.make_async_copy(tab_hbm.at[0, pl.ds(base, S_NODE)],
                                       acc, sema)
        acc_cp.start()

        def tcopy(t, s):
            return pltpu.make_async_copy(tab_hbm.at[t, pl.ds(base, S_NODE)],
                                         tmp.at[s], semm.at[s])

        tcopy(1, 1).start()
        tcopy(2, 2).start()
        tcopy(3, 3).start()
        acc_cp.wait()

        def step(t, _):
            s = lax.rem(t, 4)

            @pl.when(t + 3 < NW)
            def _():
                tcopy(t + 3, lax.rem(t + 3, 4)).start()

            tcopy(t, s).wait()

            def mx(k, _):
                sl = pl.ds(k * L, L)
                acc[sl] = jnp.maximum(acc[sl], tmp[s, sl])
                return 0

            lax.fori_loop(0, S_NODE // L, mx, 0)
            return 0

        lax.fori_loop(1, NW, step, 0)

    def clamp_into(acc, ecl):
        def clampk(k, _):
            def clampc(cc, _):
                ecl[k, pl.ds(cc * L, L)] = jnp.clip(
                    acc[pl.ds(k * GK + cc * L, L)], 0, N_EDGES - 1)
                return 0

            lax.fori_loop(0, GK // L, clampc, 0)
            return 0

        lax.fori_loop(0, S_NODE // GK, clampk, 0)

    merge(acc1, t1_hbm)
    clamp_into(acc1, ecl1)
    merge(acc2, t2_hbm)
    clamp_into(acc2, ecl2)

    NGK = S_NODE // GK

    def farb_copies(k, s, ecl, far_hbm, jfar, bwbuf, sem):
        sl = pl.ds(k * GK, GK)
        return (
            pltpu.make_async_copy(far_hbm.at[ecl.at[k]], jfar.at[k], sem.at[s]),
            pltpu.make_async_copy(b_hbm.at[ecl.at[k]], bwbuf.at[sl], sem.at[s]),
        )

    def u_copy(k, s, jfar, rowsbuf, sem):
        sl = pl.ds(k * GK, GK)
        return pltpu.make_async_copy(u_hbm.at[jfar.at[k]], rowsbuf.at[sl],
                                     sem.at[s])

    def fb1(k, s):
        return farb_copies(k, s, ecl1, i2_hbm, jfar1, bw1, semfb1)

    def fb2(k, s):
        return farb_copies(k, s, ecl2, i1_hbm, jfar2, bw2, semfb2)

    def uc1(k, s):
        return u_copy(k, s, jfar1, rowsf1, semu1)

    def uc2(k, s):
        return u_copy(k, s, jfar2, rowsf2, semu2)

    for d in fb1(0, 0) + fb2(0, 0) + fb1(1, 1) + fb2(1, 1):
        d.start()

    def gloop(k, _):
        s = lax.rem(k, 2)
        for d in fb1(k, s) + fb2(k, s):
            d.wait()

        @pl.when(k + 2 < NGK)
        def _():
            for d in fb1(k + 2, s) + fb2(k + 2, s):
                d.start()

        @pl.when(k >= 2)
        def _():
            uc1(k - 2, s).wait()
            uc2(k - 2, s).wait()

        uc1(k, s).start()
        uc2(k, s).start()
        return 0

    lax.fori_loop(0, NGK, gloop, 0)
    for k in (NGK - 2, NGK - 1):
        uc1(k, k % 2).wait()
        uc2(k, k % 2).wait()
    ubuf_cp.wait()

    io = _iota()

    def group(g, _):
        sl = pl.ds(g * L, L)
        m1 = acc1[sl] >= 0
        m2 = acc2[sl] >= 0
        eb1 = jnp.exp(-bw1[sl])
        eb2 = jnp.exp(-bw2[sl])
        rowvec = g * L + io
        for c in range(U):
            colvec = jnp.full((L,), c, jnp.int32)
            x = plsc.load_gather(ubuf, [rowvec, colvec])
            u2 = plsc.load_gather(rowsf1, [rowvec, colvec])
            u1 = plsc.load_gather(rowsf2, [rowvec, colvec])
            exn = jnp.exp(-x)
            exp_ = jnp.exp(x)
            den1 = exn + eb1 + jnp.exp(u2)
            d1 = jnp.where(m1, -(wbl[c] * exn / den1), 0.0)
            den2 = jnp.exp(-u1) + eb2 + exp_
            d2 = jnp.where(m2, wbl[c] * exp_ / den2, 0.0)
            plsc.store_scatter(ubuf, [rowvec, colvec], x + d1 + d2)
        return 0

    lax.fori_loop(0, S_NODE // L, group, 0)
    pltpu.sync_copy(ubuf, uout_hbm.at[pl.ds(base, S_NODE)])


def kernel(unary, binary, index1, index2, w_unary, w_binary):
    u_pad = _k1(unary, w_unary)
    bout, t1, t2 = _k2(u_pad, binary, index1, index2, w_binary)
    uout_pad = _k3(u_pad, binary, index1, index2, w_binary, t1, t2)
    return uout_pad[:N_NODES], bout


# speedup vs baseline: 1.4590x; 1.0004x over previous
import functools

import jax
import jax.numpy as jnp
from jax import lax
from jax.experimental import pallas as pl
from jax.experimental.pallas import tpu as pltpu
from jax.experimental.pallas import tpu_sc as plsc

N_NODES = 50000
N_EDGES = 800000
U = 16
L = 16
NC, NS = 2, 16
NW = NC * NS
S_NODE = 1568
N_PAD = NW * S_NODE
E_W = N_EDGES // NW
EK = 128
N_FULL = E_W // EK
E_TAIL = E_W - N_FULL * EK

_mesh = plsc.VectorSubcoreMesh(core_axis_name="c", subcore_axis_name="s")


def _wid():
    return lax.axis_index("s") * NC + lax.axis_index("c")


def _iota():
    return lax.iota(jnp.int32, L)


def _vperm(x, idx):
    return x.at[idx].get(mode="promise_in_bounds")


@functools.partial(
    pl.kernel,
    mesh=_mesh,
    compiler_params=pltpu.CompilerParams(needs_layout_passes=False, use_tc_tiling_on_sc=False),
    out_type=jax.ShapeDtypeStruct((N_PAD, U), jnp.float32),
    scratch_types=[
        pltpu.VMEM((2, 112, U), jnp.float32),
        pltpu.VMEM((2, 112, U), jnp.float32),
        pltpu.VMEM((L,), jnp.float32),
        pltpu.SemaphoreType.DMA((2,)),
        pltpu.SemaphoreType.DMA((2,)),
    ],
)
def _k1(unary_hbm, wu_hbm, u_hbm, inbuf, outbuf, wubuf, semi, semo):
    wid = _wid()
    io = _iota()
    wubuf[...] = jnp.zeros((L,), jnp.float32)
    pltpu.sync_copy(wu_hbm, wubuf.at[pl.ds(0, 4)])
    wuv = wubuf[...]
    wulane = _vperm(wuv, io >> 1)
    evenlane = (io & 1) == 0
    active = io < 8
    sgn_a = jnp.where(active & evenlane, -1.0, jnp.where(active, 1.0, 0.0))
    sgn_d = sgn_a
    perm = jnp.where(active, io ^ 1, io)
    base = wid * S_NODE

    CH = 112
    NCH = S_NODE // CH

    def start_of(c):
        return jnp.minimum(base + c * CH, N_NODES - CH)

    def in_cp(c, s):
        return pltpu.make_async_copy(unary_hbm.at[pl.ds(start_of(c), CH)],
                                     inbuf.at[s], semi.at[s])

    def out_cp(c, s):
        return pltpu.make_async_copy(outbuf.at[s],
                                     u_hbm.at[pl.ds(start_of(c), CH)],
                                     semo.at[s])

    in_cp(0, 0).start()
    in_cp(1, 1).start()

    def chunk(c, _):
        s = lax.rem(c, 2)
        in_cp(c, s).wait()

        @pl.when(c >= 2)
        def _():
            out_cp(c - 2, s).wait()

        def row(r, _):
            x = inbuf[s, r]
            e = jnp.exp(sgn_a * x)
            esw = _vperm(e, perm)
            delta = sgn_d * wulane * e / (e + esw)
            outbuf[s, r] = x + delta
            return 0

        lax.fori_loop(0, CH, row, 0)
        out_cp(c, s).start()

        @pl.when(c + 2 < NCH)
        def _():
            in_cp(c + 2, s).start()

        return 0

    lax.fori_loop(0, NCH, chunk, 0)
    out_cp(NCH - 2, (NCH - 2) % 2).wait()
    out_cp(NCH - 1, (NCH - 1) % 2).wait()


@functools.partial(
    pl.kernel,
    mesh=_mesh,
    compiler_params=pltpu.CompilerParams(needs_layout_passes=False, use_tc_tiling_on_sc=False),
    out_type=[
        jax.ShapeDtypeStruct((N_EDGES,), jnp.float32),
        jax.ShapeDtypeStruct((NW, N_PAD), jnp.int32),
        jax.ShapeDtypeStruct((NW, N_PAD), jnp.int32),
    ],
    scratch_types=[
        pltpu.VMEM((N_PAD,), jnp.int32),
        pltpu.VMEM((N_PAD,), jnp.int32),
        pltpu.VMEM((2, EK), jnp.int32),
        pltpu.VMEM((2, EK), jnp.int32),
        pltpu.VMEM((2, EK), jnp.float32),
        pltpu.VMEM((2, EK), jnp.float32),
        pltpu.VMEM((2, EK, U), jnp.float32),
        pltpu.VMEM((2, EK, U), jnp.float32),
        pltpu.VMEM((L,), jnp.float32),
        pltpu.SemaphoreType.DMA((2,)),
        pltpu.SemaphoreType.DMA((2,)),
        pltpu.SemaphoreType.DMA((2,)),
    ],
)
def _k2(u_hbm, b_hbm, i1_hbm, i2_hbm, wb_hbm, bout_hbm, t1_hbm, t2_hbm,
        t1, t2, i1c, i2c, bc, boutc, rows1, rows2, wbbuf,
        semi, semg, semo):
    wid = _wid()
    io = _iota()
    pltpu.sync_copy(wb_hbm, wbbuf)
    wblv = wbbuf[...]
    ebase = wid * E_W

    def in_copies(c, s):
        off = ebase + c * EK
        return (
            pltpu.make_async_copy(i1_hbm.at[pl.ds(off, EK)], i1c.at[s], semi.at[s]),
            pltpu.make_async_copy(i2_hbm.at[pl.ds(off, EK)], i2c.at[s], semi.at[s]),
            pltpu.make_async_copy(b_hbm.at[pl.ds(off, EK)], bc.at[s], semi.at[s]),
        )

    def gather_copies(s):
        return (
            pltpu.make_async_copy(u_hbm.at[i1c.at[s]], rows1.at[s], semg.at[s]),
            pltpu.make_async_copy(u_hbm.at[i2c.at[s]], rows2.at[s], semg.at[s]),
        )

    def out_copy(c, s):
        off = ebase + c * EK
        return pltpu.make_async_copy(boutc.at[s], bout_hbm.at[pl.ds(off, EK)],
                                     semo.at[s])

    def do_group(t, off, n_valid, s):
        b16 = bc[s, pl.ds(t * L, L)]
        eb = jnp.exp(-b16)
        rowidx = t * L + io
        acc = jnp.zeros((L,), jnp.float32)
        for c in range(U):
            colidx = jnp.full((L,), c, jnp.int32)
            v1 = plsc.load_gather(rows1.at[s], [rowidx, colidx])
            v2 = plsc.load_gather(rows2.at[s], [rowidx, colidx])
            d = jnp.exp(-v1) + eb + jnp.exp(v2)
            acc = acc + wblv[c] / d
        boutc[s, pl.ds(t * L, L)] = b16 - eb * acc
        eids = off + t * L + io
        for idxc, tab in ((i1c, t1), (i2c, t2)):
            nid = idxc[s, pl.ds(t * L, L)]
            key = nid * L + io
            ks, vs = plsc.sort_key_val(key, eids)
            nxt = _vperm(ks, jnp.minimum(io + 1, L - 1))
            last = ((ks >> 4) != (nxt >> 4)) | (io == L - 1)
            vmask = (ks & (L - 1)) < (n_valid - t * L)
            plsc.store_scatter(tab, [ks >> 4], vs, mask=last & vmask)

    for d in in_copies(0, 0) + in_copies(1, 1):
        d.start()

    def initt(i, _):
        neg1 = jnp.full((L,), -1, jnp.int32)
        t1[pl.ds(i * L, L)] = neg1
        t2[pl.ds(i * L, L)] = neg1
        return 0

    lax.fori_loop(0, N_PAD // L, initt, 0)

    for d in in_copies(0, 0):
        d.wait()
    for d in gather_copies(0):
        d.start()

    def pipe_chunk(c, _):
        s = lax.rem(c, 2)
        sn = 1 - s
        for d in gather_copies(s):
            d.wait()

        @pl.when(c + 1 < N_FULL)
        def _():
            for d in in_copies(c + 1, sn):
                d.wait()
            for d in gather_copies(sn):
                d.start()

        @pl.when(c >= 2)
        def _():
            out_copy(c - 2, s).wait()

        off = ebase + c * EK

        def grp(t, _):
            do_group(t, off, EK, s)
            return 0

        lax.fori_loop(0, EK // L, grp, 0)
        out_copy(c, s).start()

        @pl.when(c + 2 < N_FULL)
        def _():
            for d in in_copies(c + 2, s):
                d.start()

        return 0

    lax.fori_loop(0, N_FULL, pipe_chunk, 0)
    out_copy(N_FULL - 2, (N_FULL - 2) % 2).wait()
    out_copy(N_FULL - 1, (N_FULL - 1) % 2).wait()

    toff = ebase + N_FULL * EK
    pltpu.sync_copy(i1_hbm.at[pl.ds(toff, E_TAIL)], i1c.at[0, pl.ds(0, E_TAIL)])
    pltpu.sync_copy(i2_hbm.at[pl.ds(toff, E_TAIL)], i2c.at[0, pl.ds(0, E_TAIL)])
    pltpu.sync_copy(b_hbm.at[pl.ds(toff, E_TAIL)], bc.at[0, pl.ds(0, E_TAIL)])
    pltpu.sync_copy(u_hbm.at[i1c.at[0]], rows1.at[0])
    pltpu.sync_copy(u_hbm.at[i2c.at[0]], rows2.at[0])

    def tgrp(t, _):
        do_group(t, toff, E_TAIL, 0)
        return 0

    lax.fori_loop(0, (E_TAIL + L - 1) // L, tgrp, 0)
    pltpu.sync_copy(boutc.at[0, pl.ds(0, E_TAIL)],
                    bout_hbm.at[pl.ds(toff, E_TAIL)])

    pltpu.sync_copy(t1, t1_hbm.at[wid])
    pltpu.sync_copy(t2, t2_hbm.at[wid])


GK = 112


@functools.partial(
    pl.kernel,
    mesh=_mesh,
    compiler_params=pltpu.CompilerParams(needs_layout_passes=False, use_tc_tiling_on_sc=False),
    out_type=jax.ShapeDtypeStruct((N_PAD, U), jnp.float32),
    scratch_types=[
        pltpu.VMEM((S_NODE,), jnp.int32),
        pltpu.VMEM((S_NODE,), jnp.int32),
        pltpu.VMEM((4, S_NODE), jnp.int32),
        pltpu.VMEM((S_NODE // GK, GK), jnp.int32),
        pltpu.VMEM((S_NODE // GK, GK), jnp.int32),
        pltpu.VMEM((S_NODE // GK, GK), jnp.int32),
        pltpu.VMEM((S_NODE // GK, GK), jnp.int32),
        pltpu.VMEM((S_NODE,), jnp.float32),
        pltpu.VMEM((S_NODE,), jnp.float32),
        pltpu.VMEM((S_NODE, U), jnp.float32),
        pltpu.VMEM((S_NODE, U), jnp.float32),
        pltpu.VMEM((S_NODE, U), jnp.float32),
        pltpu.VMEM((L,), jnp.float32),
        pltpu.SemaphoreType.DMA,
        pltpu.SemaphoreType.DMA,
        pltpu.SemaphoreType.DMA((4,)),
        pltpu.SemaphoreType.DMA((2,)),
        pltpu.SemaphoreType.DMA((2,)),
        pltpu.SemaphoreType.DMA((2,)),
        pltpu.SemaphoreType.DMA((2,)),
    ],
)
def _k3(u_hbm, b_hbm, i1_hbm, i2_hbm, wb_hbm, t1_hbm, t2_hbm, uout_hbm,
        acc1, acc2, tmp, ecl1, ecl2, jfar1, jfar2, bw1, bw2,
        rowsf1, rowsf2, ubuf, wbbuf,
        semus, sema, semm, semfb1, semfb2, semu1, semu2):
    wid = _wid()
    base = wid * S_NODE
    ubuf_cp = pltpu.make_async_copy(u_hbm.at[pl.ds(base, S_NODE)], ubuf, semus)
    ubuf_cp.start()
    pltpu.sync_copy(wb_hbm, wbbuf)
    wbl = wbbuf[...]

    def merge(acc, tab_hbm):
        acc_cp = pltpu.make_async_copy(tab_hbm.at[0, pl.ds(base, S_NODE)],
                                       acc, sema)
        acc_cp.start()

        def tcopy(t, s):
            return pltpu.make_async_copy(tab_hbm.at[t, pl.ds(base, S_NODE)],
                                         tmp.at[s], semm.at[s])

        tcopy(1, 1).start()
        tcopy(2, 2).start()
        tcopy(3, 3).start()
        acc_cp.wait()

        def step(t, _):
            s = lax.rem(t, 4)

            @pl.when(t + 3 < NW)
            def _():
                tcopy(t + 3, lax.rem(t + 3, 4)).start()

            tcopy(t, s).wait()

            def mx(k, _):
                sl = pl.ds(k * L, L)
                acc[sl] = jnp.maximum(acc[sl], tmp[s, sl])
                return 0

            lax.fori_loop(0, S_NODE // L, mx, 0)
            return 0

        lax.fori_loop(1, NW, step, 0)

    def clamp_into(acc, ecl):
        def clampk(k, _):
            def clampc(cc, _):
                ecl[k, pl.ds(cc * L, L)] = jnp.clip(
                    acc[pl.ds(k * GK + cc * L, L)], 0, N_EDGES - 1)
                return 0

            lax.fori_loop(0, GK // L, clampc, 0)
            return 0

        lax.fori_loop(0, S_NODE // GK, clampk, 0)

    merge(acc1, t1_hbm)
    clamp_into(acc1, ecl1)
    merge(acc2, t2_hbm)
    clamp_into(acc2, ecl2)

    NGK = S_NODE // GK

    def farb_copies(k, s, ecl, far_hbm, jfar, bwbuf, sem):
        sl = pl.ds(k * GK, GK)
        return (
            pltpu.make_async_copy(far_hbm.at[ecl.at[k]], jfar.at[k], sem.at[s]),
            pltpu.make_async_copy(b_hbm.at[ecl.at[k]], bwbuf.at[sl], sem.at[s]),
        )

    def u_copy(k, s, jfar, rowsbuf, sem):
        sl = pl.ds(k * GK, GK)
        return pltpu.make_async_copy(u_hbm.at[jfar.at[k]], rowsbuf.at[sl],
                                     sem.at[s])

    def fb1(k, s):
        return farb_copies(k, s, ecl1, i2_hbm, jfar1, bw1, semfb1)

    def fb2(k, s):
        return farb_copies(k, s, ecl2, i1_hbm, jfar2, bw2, semfb2)

    def uc1(k, s):
        return u_copy(k, s, jfar1, rowsf1, semu1)

    def uc2(k, s):
        return u_copy(k, s, jfar2, rowsf2, semu2)

    for d in fb1(0, 0) + fb2(0, 0) + fb1(1, 1) + fb2(1, 1):
        d.start()

    def gloop(k, _):
        s = lax.rem(k, 2)
        for d in fb1(k, s) + fb2(k, s):
            d.wait()

        @pl.when(k + 2 < NGK)
        def _():
            for d in fb1(k + 2, s) + fb2(k + 2, s):
                d.start()

        @pl.when(k >= 2)
        def _():
            uc1(k - 2, s).wait()
            uc2(k - 2, s).wait()

        uc1(k, s).start()
        uc2(k, s).start()
        return 0

    lax.fori_loop(0, NGK, gloop, 0)
    for k in (NGK - 2, NGK - 1):
        uc1(k, k % 2).wait()
        uc2(k, k % 2).wait()
    ubuf_cp.wait()

    io = _iota()

    def group(g, _):
        sl = pl.ds(g * L, L)
        m1 = acc1[sl] >= 0
        m2 = acc2[sl] >= 0
        eb1 = jnp.exp(-bw1[sl])
        eb2 = jnp.exp(-bw2[sl])
        rowvec = g * L + io
        for c in range(U):
            colvec = jnp.full((L,), c, jnp.int32)
            x = plsc.load_gather(ubuf, [rowvec, colvec])
            u2 = plsc.load_gather(rowsf1, [rowvec, colvec])
            u1 = plsc.load_gather(rowsf2, [rowvec, colvec])
            exn = jnp.exp(-x)
            exp_ = jnp.exp(x)
            den1 = exn + eb1 + jnp.exp(u2)
            d1 = jnp.where(m1, -(wbl[c] * exn / den1), 0.0)
            den2 = jnp.exp(-u1) + eb2 + exp_
            d2 = jnp.where(m2, wbl[c] * exp_ / den2, 0.0)
            plsc.store_scatter(ubuf, [rowvec, colvec], x + d1 + d2)
        return 0

    lax.fori_loop(0, S_NODE // L, group, 0)
    pltpu.sync_copy(ubuf, uout_hbm.at[pl.ds(base, S_NODE)])


def kernel(unary, binary, index1, index2, w_unary, w_binary):
    u_pad = _k1(unary, w_unary)
    bout, t1, t2 = _k2(u_pad, binary, index1, index2, w_binary)
    uout_pad = _k3(u_pad, binary, index1, index2, w_binary, t1, t2)
    return uout_pad[:N_NODES], bout
